# baseline (speedup 1.0000x reference)
import functools

import jax
import jax.numpy as jnp
from jax import lax
from jax.experimental import pallas as pl
from jax.experimental.pallas import tpu as pltpu
from jax.experimental.pallas import tpu_sc as plsc

NC = 2
NS = 16
NW = NC * NS

NBIN = 8
NBOUT = 2
L = 16


def kernel(x, token_table, pos_table):
    batch, seq = x.shape
    vocab, embed = token_table.shape
    assert embed % L == 0
    bcw = batch // NW
    assert bcw * NW == batch and bcw % L == 0 and bcw <= 128
    assert seq % NBIN == 0 and NBIN % NBOUT == 0

    x_t = x.T.astype(jnp.int32)

    mesh = plsc.VectorSubcoreMesh(core_axis_name="c", subcore_axis_name="s")

    @functools.partial(
        pl.kernel,
        mesh=mesh,
        compiler_params=pltpu.CompilerParams(
            use_tc_tiling_on_sc=False, needs_layout_passes=False
        ),
        out_type=jax.ShapeDtypeStruct((seq, embed, batch), jnp.float32),
        scratch_types=(
            [pltpu.VMEM((seq, bcw), jnp.int32),
             pltpu.VMEM((seq, embed), jnp.float32)]
            + [pltpu.VMEM((bcw, embed), jnp.float32) for _ in range(NBIN)]
            + [pltpu.VMEM((embed, bcw + 1), jnp.float32) for _ in range(NBOUT)]
            + [pltpu.SemaphoreType.DMA for _ in range(NBIN + NBOUT)]
        ),
    )
    def emb(x_hbm, tok_hbm, pos_hbm, out_hbm, idx_v, pos_v, *bufs_sems):
        inb = bufs_sems[:NBIN]
        outb = bufs_sems[NBIN:NBIN + NBOUT]
        gsem = bufs_sems[NBIN + NBOUT:2 * NBIN + NBOUT]
        ssem = bufs_sems[2 * NBIN + NBOUT:]
        wid = lax.axis_index("s") * NC + lax.axis_index("c")
        col0 = wid * bcw
        ci = lax.iota(jnp.int32, L)

        pltpu.sync_copy(pos_hbm, pos_v)
        pltpu.sync_copy(x_hbm.at[:, pl.ds(col0, bcw)], idx_v)

        def fire_gather(s, b):
            pltpu.async_copy(tok_hbm.at[idx_v.at[s]], inb[b], gsem[b])

        def drain_gather(b):
            pltpu.make_async_copy(tok_hbm.at[pl.ds(0, bcw)], inb[b], gsem[b]).wait()

        def wait_store(b):
            pltpu.make_async_copy(
                outb[b].at[:, pl.ds(0, bcw)],
                out_hbm.at[0, :, pl.ds(col0, bcw)],
                ssem[b],
            ).wait()

        def compute(s, bi, bo):
            pc = [pos_v[s, pl.ds(q * L, L)] for q in range(embed // L)]

            def body(r2, c):
                for u in range(2):
                    r = r2 * 2 + u
                    rs = jnp.full((L,), r, jnp.int32)
                    for q in range(embed // L):
                        val = inb[bi][r, pl.ds(q * L, L)] + pc[q]
                        plsc.store_scatter(outb[bo], [ci + q * L, rs], val)
                return c

            lax.fori_loop(0, bcw // 2, body, 0)

        def slot(s, bi, bo):
            drain_gather(bi)

            @pl.when(s >= NBOUT)
            def _():
                wait_store(bo)

            compute(s, bi, bo)
            pltpu.async_copy(
                outb[bo].at[:, pl.ds(0, bcw)],
                out_hbm.at[s, :, pl.ds(col0, bcw)],
                ssem[bo],
            )

            @pl.when(s + NBIN < seq)
            def _():
                fire_gather(s + NBIN, bi)

        for s0 in range(NBIN):
            fire_gather(s0, s0)

        def outer(o, c):
            for p in range(NBIN):
                slot(o * NBIN + p, p, p % NBOUT)
            return c
        lax.fori_loop(0, seq // NBIN, outer, 0)

        for b in range(NBOUT):
            wait_store(b)

    o3 = emb(x_t, token_table, pos_table)
    return jnp.transpose(o3, (2, 0, 1))


# speedup vs baseline: 1.2278x; 1.2278x over previous
import functools

import jax
import jax.numpy as jnp
from jax import lax
from jax.experimental import pallas as pl
from jax.experimental.pallas import tpu as pltpu
from jax.experimental.pallas import tpu_sc as plsc

NC = 2
NS = 16
NW = NC * NS

NBIN = 8
NBOUT = 2
L = 16


def kernel(x, token_table, pos_table):
    batch, seq = x.shape
    vocab, embed = token_table.shape
    assert embed == 64
    bcw = batch // NW
    assert bcw * NW == batch and bcw == 128
    assert seq % NBIN == 0 and NBIN % NBOUT == 0
    eb = embed // 8

    x_t = x.T.astype(jnp.int32)

    mesh = plsc.VectorSubcoreMesh(core_axis_name="c", subcore_axis_name="s")

    @functools.partial(
        pl.kernel,
        mesh=mesh,
        compiler_params=pltpu.CompilerParams(
            use_tc_tiling_on_sc=False, needs_layout_passes=False
        ),
        out_type=jax.ShapeDtypeStruct((seq, eb, NW, 8, bcw), jnp.float32),
        scratch_types=(
            [pltpu.VMEM((seq, bcw), jnp.int32),
             pltpu.VMEM((seq, embed), jnp.float32)]
            + [pltpu.VMEM((bcw, embed), jnp.float32) for _ in range(NBIN)]
            + [pltpu.VMEM((eb, 8, bcw + 1), jnp.float32) for _ in range(NBOUT)]
            + [pltpu.SemaphoreType.DMA for _ in range(NBIN + NBOUT)]
        ),
    )
    def emb(x_hbm, tok_hbm, pos_hbm, out_hbm, idx_v, pos_v, *bufs_sems):
        inb = bufs_sems[:NBIN]
        outb = bufs_sems[NBIN:NBIN + NBOUT]
        gsem = bufs_sems[NBIN + NBOUT:2 * NBIN + NBOUT]
        ssem = bufs_sems[2 * NBIN + NBOUT:]
        wid = lax.axis_index("s") * NC + lax.axis_index("c")
        col0 = wid * bcw
        ci = lax.iota(jnp.int32, L)
        lmod = lax.rem(ci, jnp.full((L,), 8, jnp.int32))
        aq = [lax.div(ci, jnp.full((L,), 8, jnp.int32)) + 2 * q
              for q in range(embed // L)]

        pltpu.sync_copy(pos_hbm, pos_v)
        pltpu.sync_copy(x_hbm.at[:, pl.ds(col0, bcw)], idx_v)

        def fire_gather(s, b):
            pltpu.async_copy(tok_hbm.at[idx_v.at[s]], inb[b], gsem[b])

        def drain_gather(b):
            pltpu.make_async_copy(tok_hbm.at[pl.ds(0, bcw)], inb[b], gsem[b]).wait()

        def wait_store(b):
            pltpu.make_async_copy(
                outb[b].at[:, :, pl.ds(0, bcw)],
                out_hbm.at[0, :, 0],
                ssem[b],
            ).wait()

        def compute(s, bi, bo):
            pc = [pos_v[s, pl.ds(q * L, L)] for q in range(embed // L)]

            def body(r4, c):
                for u in range(4):
                    r = r4 * 4 + u
                    rs = jnp.full((L,), r, jnp.int32)
                    for q in range(embed // L):
                        val = inb[bi][r, pl.ds(q * L, L)] + pc[q]
                        plsc.store_scatter(outb[bo], [aq[q], lmod, rs], val)
                return c

            lax.fori_loop(0, bcw // 4, body, 0)

        def slot(s, bi, bo):
            drain_gather(bi)

            @pl.when(s >= NBOUT)
            def _():
                wait_store(bo)

            compute(s, bi, bo)
            pltpu.async_copy(
                outb[bo].at[:, :, pl.ds(0, bcw)],
                out_hbm.at[s, :, wid],
                ssem[bo],
            )

            @pl.when(s + NBIN < seq)
            def _():
                fire_gather(s + NBIN, bi)

        for s0 in range(NBIN):
            fire_gather(s0, s0)

        def outer(o, c):
            for p in range(NBIN):
                slot(o * NBIN + p, p, p % NBOUT)
            return c
        lax.fori_loop(0, seq // NBIN, outer, 0)

        for b in range(NBOUT):
            wait_store(b)

    o5 = emb(x_t, token_table, pos_table)
    return o5.transpose(2, 4, 0, 1, 3).reshape(batch, seq, embed)
